# baseline (speedup 1.0000x reference)
import functools

import jax
import jax.numpy as jnp
from jax import lax
from jax.experimental import pallas as pl
from jax.experimental.pallas import tpu as pltpu
from jax.experimental.pallas import tpu_sc as plsc

N_OBJ = 1000
N_REL = 5000
NUM_OBJ_CLS = 151
NUM_REL_CLS = 51
OBJ_DIM = 4096
REL_DIM = 4096
HID = 512
T_STEPS = 3
NMS_THRESH = 0.3

NP_OBJ = 1024
CP = 256
NP_REL = 5120
RB = 512
NEG = -1e30


def _probs_body(logits_ref, out_ref):
    x = logits_ref[...]
    m = jnp.max(x, axis=1, keepdims=True)
    e = jnp.exp(x - m)
    out_ref[...] = e / jnp.sum(e, axis=1, keepdims=True)


def _probs_call(logits_p):
    return pl.pallas_call(
        _probs_body,
        out_shape=jax.ShapeDtypeStruct((N_OBJ, CP), jnp.float32),
    )(logits_p)


def _nms_body(x1r, y1r, x2r, y2r, sr, x1c, y1c, x2c, y2c, sc, keep_ref):
    x1_r = x1r[0]
    y1_r = y1r[0]
    x2_r = x2r[0]
    y2_r = y2r[0]
    s_r = sr[0]
    x1_c = x1c[0]
    y1_c = y1c[0]
    x2_c = x2c[0]
    y2_c = y2c[0]
    s_c = sc[0]

    area_r = (x2_r - x1_r) * (y2_r - y1_r)
    area_c = (x2_c - x1_c) * (y2_c - y1_c)
    xx1 = jnp.maximum(x1_c, x1_r)
    yy1 = jnp.maximum(y1_c, y1_r)
    xx2 = jnp.minimum(x2_c, x2_r)
    yy2 = jnp.minimum(y2_c, y2_r)
    w = jnp.maximum(xx2 - xx1, 0.0)
    h = jnp.maximum(yy2 - yy1, 0.0)
    inter = w * h
    iou = inter / (area_c + area_r - inter + 1e-8)

    ii = lax.broadcasted_iota(jnp.int32, (NP_OBJ, NP_OBJ), 0)
    jj = lax.broadcasted_iota(jnp.int32, (NP_OBJ, NP_OBJ), 1)
    prec = (s_c > s_r) | ((s_c == s_r) & (ii < jj))
    q = ((iou > NMS_THRESH) & prec & (ii < N_OBJ) & (jj < N_OBJ)
         ).astype(jnp.float32)

    lane = lax.broadcasted_iota(jnp.int32, (1, NP_OBJ), 1)
    valid = (lane < N_OBJ).astype(jnp.float32)

    def cond(state):
        _, _, und = state
        return jnp.sum(und) > 0.0

    def body(state):
        kept, supp, und = state
        blocked = jnp.dot(kept, q, preferred_element_type=jnp.float32)
        new_supp = und * (blocked > 0.0).astype(jnp.float32)
        supp = supp + new_supp
        und = und - new_supp
        notsupp = valid - supp
        pending = jnp.dot(notsupp, q, preferred_element_type=jnp.float32)
        new_kept = und * (pending == 0.0).astype(jnp.float32)
        kept = kept + new_kept
        und = und - new_kept
        return kept, supp, und

    z = jnp.zeros((1, NP_OBJ), jnp.float32)
    kept, _, _ = lax.while_loop(cond, body, (z, 1.0 - valid, valid))
    keep_ref[...] = kept[None]


def _nms_call(rows, cols):
    nclass = NUM_OBJ_CLS - 1
    row_spec = pl.BlockSpec((1, 1, NP_OBJ), lambda c: (c, 0, 0))
    col_spec = pl.BlockSpec((1, NP_OBJ, 1), lambda c: (c, 0, 0))
    return pl.pallas_call(
        _nms_body,
        grid=(nclass,),
        in_specs=[row_spec] * 5 + [col_spec] * 5,
        out_specs=pl.BlockSpec((1, 1, NP_OBJ), lambda c: (c, 0, 0)),
        out_shape=jax.ShapeDtypeStruct((nclass, 1, NP_OBJ), jnp.float32),
    )(*rows, *cols)


def _relmm_body(x_ref, w_ref, b_ref, o_ref):
    o_ref[...] = (
        jnp.dot(x_ref[...], w_ref[...], preferred_element_type=jnp.float32)
        + b_ref[...]
    )


def _relmm_call(vr, w, b):
    mb = 200
    return pl.pallas_call(
        _relmm_body,
        grid=(N_REL // mb,),
        in_specs=[
            pl.BlockSpec((mb, REL_DIM), lambda i: (i, 0)),
            pl.BlockSpec((REL_DIM, HID), lambda i: (0, 0)),
            pl.BlockSpec((1, HID), lambda i: (0, 0)),
        ],
        out_specs=pl.BlockSpec((mb, HID), lambda i: (i, 0)),
        out_shape=jax.ShapeDtypeStruct((N_REL, HID), jnp.float32),
    )(vr, w, b)


def _objmm_body(f_ref, w_ref, b_ref, lg_ref, mask_ref, probs_ref, wemb_ref,
                h_ref, pred_ref):
    lg = lg_ref[...]
    mask = mask_ref[...]
    lane = lax.broadcasted_iota(jnp.int32, lg.shape, 1)
    lp = mask * lg + (1.0 - mask) * (-1000.0)
    lp = jnp.where(lane < NUM_OBJ_CLS, lp, NEG)
    m = jnp.max(lp, axis=1, keepdims=True)
    e = jnp.exp(lp - m)
    p2 = e / jnp.sum(e, axis=1, keepdims=True)
    emb = jnp.dot(p2, wemb_ref[...], preferred_element_type=jnp.float32)
    h = jnp.dot(f_ref[...], w_ref[...], preferred_element_type=jnp.float32)
    h_ref[...] = jnp.tanh(h + b_ref[...] + emb)

    mp = mask * probs_ref[...]
    adj = jnp.where((lane >= 1) & (lane < NUM_OBJ_CLS), mp, -1.0)
    mx = jnp.max(adj, axis=1, keepdims=True)
    cand = jnp.where(adj == mx, lane, NUM_OBJ_CLS + 1)
    pred_ref[...] = jnp.min(cand, axis=1, keepdims=True)


def _objmm_call(fmaps, w, b, logits_p, mask_p, probs, wemb_p):
    mb = 200
    return pl.pallas_call(
        _objmm_body,
        grid=(N_OBJ // mb,),
        in_specs=[
            pl.BlockSpec((mb, OBJ_DIM), lambda i: (i, 0)),
            pl.BlockSpec((OBJ_DIM, HID), lambda i: (0, 0)),
            pl.BlockSpec((1, HID), lambda i: (0, 0)),
            pl.BlockSpec((mb, CP), lambda i: (i, 0)),
            pl.BlockSpec((mb, CP), lambda i: (i, 0)),
            pl.BlockSpec((mb, CP), lambda i: (i, 0)),
            pl.BlockSpec((CP, HID), lambda i: (0, 0)),
        ],
        out_specs=[
            pl.BlockSpec((mb, HID), lambda i: (i, 0)),
            pl.BlockSpec((mb, 1), lambda i: (i, 0)),
        ],
        out_shape=[
            jax.ShapeDtypeStruct((N_OBJ, HID), jnp.float32),
            jax.ShapeDtypeStruct((N_OBJ, 1), jnp.int32),
        ],
    )(fmaps, w, b, logits_p, mask_p, probs, wemb_p)


CHUNK = 80
NCHUNK = NP_REL // CHUNK
_NC = 2
_NS = 16
_ROWS_PER_SUB = NP_OBJ // _NS


def _sc_gather_body(objh_hbm, sub_hbm, ob_hbm, ga_hbm, gb_hbm,
                    idx_v, rows_v, sem):
    cid = lax.axis_index("c")
    sid = lax.axis_index("s")
    wid = sid * _NC + cid
    for h in range(2):
        k = wid * 2 + h
        pltpu.sync_copy(sub_hbm.at[k], idx_v)
        pltpu.async_copy(objh_hbm.at[idx_v], rows_v, sem).wait()
        pltpu.sync_copy(rows_v, ga_hbm.at[pl.ds(k * CHUNK, CHUNK)])
        pltpu.sync_copy(ob_hbm.at[k], idx_v)
        pltpu.async_copy(objh_hbm.at[idx_v], rows_v, sem).wait()
        pltpu.sync_copy(rows_v, gb_hbm.at[pl.ds(k * CHUNK, CHUNK)])


@functools.cache
def _sc_kernels():
    mesh = plsc.VectorSubcoreMesh(core_axis_name="c", subcore_axis_name="s",
                                  num_cores=_NC)
    gather = pl.kernel(
        _sc_gather_body,
        mesh=mesh,
        out_type=[jax.ShapeDtypeStruct((NP_REL, HID), jnp.float32),
                  jax.ShapeDtypeStruct((NP_REL, HID), jnp.float32)],
        scratch_types=[
            pltpu.VMEM((CHUNK,), jnp.int32),
            pltpu.VMEM((CHUNK, HID), jnp.float32),
            pltpu.SemaphoreType.DMA,
        ],
    )
    return gather


def _segsum(vr_get, subr_ref, obr_ref, objh, wmsg_ref):
    iota_obj_row = lax.broadcasted_iota(jnp.int32, (NP_OBJ, RB), 0)
    msg = jnp.zeros((NP_OBJ, HID), jnp.float32)
    for rb in range(NP_REL // RB):
        sub_blk = subr_ref[:, rb * RB:(rb + 1) * RB]
        ob_blk = obr_ref[:, rb * RB:(rb + 1) * RB]
        oh = ((sub_blk == iota_obj_row).astype(jnp.float32)
              + (ob_blk == iota_obj_row).astype(jnp.float32))
        msg = msg + jnp.dot(oh, vr_get(rb), preferred_element_type=jnp.float32)
    nrm = jnp.sqrt(jnp.sum(msg * msg, axis=1, keepdims=True))
    msg = msg / (nrm + 1e-8)
    return jnp.tanh(
        objh + jnp.dot(msg, wmsg_ref[...], preferred_element_type=jnp.float32))


def _seg0_body(vr_ref, subr_ref, obr_ref, objh_ref, wmsg_ref, out_ref):
    out_ref[...] = _segsum(
        lambda rb: vr_ref[rb * RB:(rb + 1) * RB, :],
        subr_ref, obr_ref, objh_ref[...], wmsg_ref)


def _seg0_call(vr, subr, obr, objh, wmsg):
    return pl.pallas_call(
        _seg0_body,
        out_shape=jax.ShapeDtypeStruct((NP_OBJ, HID), jnp.float32),
    )(vr, subr, obr, objh, wmsg)


def _segup_body(vr0_ref, ga_ref, gb_ref, subr_ref, obr_ref, objh_ref,
                wmsg_ref, objh_out, vr_out):
    for rb in range(NP_REL // RB):
        s = pl.ds(rb * RB, RB)
        vr_out[s, :] = jnp.tanh(vr0_ref[s, :] + ga_ref[s, :] + gb_ref[s, :])
    objh_out[...] = _segsum(
        lambda rb: vr_out[rb * RB:(rb + 1) * RB, :],
        subr_ref, obr_ref, objh_ref[...], wmsg_ref)


def _segup_call(vr0, ga, gb, subr, obr, objh, wmsg):
    return pl.pallas_call(
        _segup_body,
        out_shape=[jax.ShapeDtypeStruct((NP_OBJ, HID), jnp.float32),
                   jax.ShapeDtypeStruct((NP_REL, HID), jnp.float32)],
    )(vr0, ga, gb, subr, obr, objh, wmsg)


def _final_body(vr_ref, ga_ref, gb_ref, wout_ref, bout_ref, out_ref):
    ga = ga_ref[...]
    gb = gb_ref[...]
    vrn = jnp.tanh(vr_ref[...] + ga + gb)
    w1 = wout_ref[0:HID, :]
    w2 = wout_ref[HID:2 * HID, :]
    w3 = wout_ref[2 * HID:3 * HID, :]
    out_ref[...] = (
        jnp.dot(ga, w1, preferred_element_type=jnp.float32)
        + jnp.dot(gb, w2, preferred_element_type=jnp.float32)
        + jnp.dot(vrn, w3, preferred_element_type=jnp.float32)
        + bout_ref[...])


def _final_call(vr, ga, gb, wout_p, bout_p):
    blk = pl.BlockSpec((RB, HID), lambda i: (i, 0))
    return pl.pallas_call(
        _final_body,
        grid=(NP_REL // RB,),
        in_specs=[blk, blk, blk,
                  pl.BlockSpec((3 * HID, 128), lambda i: (0, 0)),
                  pl.BlockSpec((1, 128), lambda i: (0, 0))],
        out_specs=pl.BlockSpec((RB, 128), lambda i: (i, 0)),
        out_shape=jax.ShapeDtypeStruct((NP_REL, 128), jnp.float32),
    )(vr, ga, gb, wout_p, bout_p)


def kernel(im_inds, obj_fmaps, obj_logits, rel_inds, vr, boxes_per_cls,
           W_obj, b_obj, W_rel, b_rel, W_emb, W_msg, W_out, b_out):
    f32 = jnp.float32
    nclass = NUM_OBJ_CLS - 1

    logits_p = jnp.pad(obj_logits, ((0, 0), (0, CP - NUM_OBJ_CLS)),
                       constant_values=NEG)
    probs = _probs_call(logits_p)

    bt = jnp.transpose(boxes_per_cls[:, 1:, :], (1, 2, 0))
    bt = jnp.pad(bt, ((0, 0), (0, 0), (0, NP_OBJ - N_OBJ)))
    st = jnp.transpose(probs[:, 1:NUM_OBJ_CLS])
    st = jnp.pad(st, ((0, 0), (0, NP_OBJ - N_OBJ)), constant_values=-1.0)
    rows = [bt[:, 0].reshape(nclass, 1, NP_OBJ),
            bt[:, 1].reshape(nclass, 1, NP_OBJ),
            bt[:, 2].reshape(nclass, 1, NP_OBJ),
            bt[:, 3].reshape(nclass, 1, NP_OBJ),
            st.reshape(nclass, 1, NP_OBJ)]
    cols = [bt[:, 0].reshape(nclass, NP_OBJ, 1),
            bt[:, 1].reshape(nclass, NP_OBJ, 1),
            bt[:, 2].reshape(nclass, NP_OBJ, 1),
            bt[:, 3].reshape(nclass, NP_OBJ, 1),
            st.reshape(nclass, NP_OBJ, 1)]
    keep = _nms_call(rows, cols)
    keep2 = jnp.transpose(keep[:, 0, :N_OBJ])
    mask = jnp.concatenate([jnp.zeros((N_OBJ, 1), f32), keep2], axis=1)
    mask_p = jnp.pad(mask, ((0, 0), (0, CP - NUM_OBJ_CLS)))

    vr_h0 = _relmm_call(vr, W_rel, b_rel.reshape(1, HID))
    wemb_p = jnp.pad(W_emb, ((0, CP - NUM_OBJ_CLS), (0, 0)))
    obj_h0, preds = _objmm_call(obj_fmaps, W_obj, b_obj.reshape(1, HID),
                                logits_p, mask_p, probs, wemb_p)

    sub = rel_inds[:, 1]
    ob = rel_inds[:, 2]
    sub2 = jnp.pad(sub, (0, NP_REL - N_REL),
                   constant_values=NP_OBJ - 1).reshape(NCHUNK, CHUNK)
    ob2 = jnp.pad(ob, (0, NP_REL - N_REL),
                  constant_values=NP_OBJ - 1).reshape(NCHUNK, CHUNK)
    vr_hp = jnp.pad(vr_h0, ((0, NP_REL - N_REL), (0, 0)))
    objh_p = jnp.pad(obj_h0, ((0, NP_OBJ - N_OBJ), (0, 0)))
    wout_p = jnp.pad(W_out, ((0, 0), (0, 128 - NUM_REL_CLS)))
    bout_p = jnp.pad(b_out, (0, 128 - NUM_REL_CLS)).reshape(1, 128)

    sc_gather = _sc_kernels()
    subr = sub2.reshape(1, NP_REL)
    obr = ob2.reshape(1, NP_REL)
    objh_p = _seg0_call(vr_hp, subr, obr, objh_p, W_msg)
    for t in range(1, T_STEPS):
        ga, gb = sc_gather(objh_p, sub2, ob2)
        objh_p, vr_hp = _segup_call(vr_hp, ga, gb, subr, obr, objh_p, W_msg)
    ga, gb = sc_gather(objh_p, sub2, ob2)
    rel_p = _final_call(vr_hp, ga, gb, wout_p, bout_p)
    rel_logits = rel_p[:N_REL, :NUM_REL_CLS]

    obj_preds = preds.reshape(N_OBJ).astype(jnp.int32)
    return (obj_logits, obj_preds, rel_logits)


# speedup vs baseline: 1.1033x; 1.1033x over previous
import functools

import jax
import jax.numpy as jnp
from jax import lax
from jax.experimental import pallas as pl
from jax.experimental.pallas import tpu as pltpu
from jax.experimental.pallas import tpu_sc as plsc

N_OBJ = 1000
N_REL = 5000
NUM_OBJ_CLS = 151
NUM_REL_CLS = 51
OBJ_DIM = 4096
REL_DIM = 4096
HID = 512
T_STEPS = 3
NMS_THRESH = 0.3

NP_OBJ = 1024
CP = 256
NP_REL = 5120
RB = 512
NEG = -1e30


def _probs_body(logits_ref, out_ref):
    x = logits_ref[...]
    m = jnp.max(x, axis=1, keepdims=True)
    e = jnp.exp(x - m)
    out_ref[...] = e / jnp.sum(e, axis=1, keepdims=True)


def _probs_call(logits_p):
    return pl.pallas_call(
        _probs_body,
        out_shape=jax.ShapeDtypeStruct((N_OBJ, CP), jnp.float32),
    )(logits_p)


CB = 10


def _nms_body(rowp_ref, colp_ref, keep_ref, qs_ref):
    ii = lax.broadcasted_iota(jnp.int32, (NP_OBJ, NP_OBJ), 0)
    jj = lax.broadcasted_iota(jnp.int32, (NP_OBJ, NP_OBJ), 1)
    for c in range(CB):
        b = c * 5
        x1_r = rowp_ref[0, b + 0:b + 1, :]
        y1_r = rowp_ref[0, b + 1:b + 2, :]
        x2_r = rowp_ref[0, b + 2:b + 3, :]
        y2_r = rowp_ref[0, b + 3:b + 4, :]
        s_r = rowp_ref[0, b + 4:b + 5, :]
        x1_c = colp_ref[0, :, b + 0:b + 1]
        y1_c = colp_ref[0, :, b + 1:b + 2]
        x2_c = colp_ref[0, :, b + 2:b + 3]
        y2_c = colp_ref[0, :, b + 3:b + 4]
        s_c = colp_ref[0, :, b + 4:b + 5]

        area_r = (x2_r - x1_r) * (y2_r - y1_r)
        area_c = (x2_c - x1_c) * (y2_c - y1_c)
        xx1 = jnp.maximum(x1_c, x1_r)
        yy1 = jnp.maximum(y1_c, y1_r)
        xx2 = jnp.minimum(x2_c, x2_r)
        yy2 = jnp.minimum(y2_c, y2_r)
        w = jnp.maximum(xx2 - xx1, 0.0)
        h = jnp.maximum(yy2 - yy1, 0.0)
        inter = w * h
        iou = inter / (area_c + area_r - inter + 1e-8)

        prec = (s_c > s_r) | ((s_c == s_r) & (ii < jj))
        qs_ref[c] = ((iou > NMS_THRESH) & prec & (ii < N_OBJ) & (jj < N_OBJ)
                     ).astype(jnp.bfloat16)

    lane = lax.broadcasted_iota(jnp.int32, (CB, NP_OBJ), 1)
    valid = (lane < N_OBJ).astype(jnp.float32)

    def cond(state):
        _, _, und = state
        return jnp.sum(und) > 0.0

    def body(state):
        kept, supp, und = state
        notsupp = valid - supp
        new_kept_rows = []
        new_supp_rows = []
        for c in range(CB):
            lhs = jnp.concatenate(
                [kept[c:c + 1], notsupp[c:c + 1]], axis=0).astype(jnp.bfloat16)
            bp = jnp.dot(lhs, qs_ref[c], preferred_element_type=jnp.float32)
            new_supp_rows.append(und[c:c + 1] * (bp[0:1] > 0.0))
            new_kept_rows.append(und[c:c + 1] * (bp[1:2] == 0.0))
        new_supp = jnp.concatenate(new_supp_rows, axis=0)
        new_kept = jnp.concatenate(new_kept_rows, axis=0)
        kept = kept + new_kept
        supp = supp + new_supp
        und = und - new_kept - new_supp
        return kept, supp, und

    z = jnp.zeros((CB, NP_OBJ), jnp.float32)
    kept, _, _ = lax.while_loop(cond, body, (z, 1.0 - valid, valid))
    keep_ref[...] = kept[:, None, :]


def _nms_call(rowp, colp):
    nclass = NUM_OBJ_CLS - 1
    nb = nclass // CB
    return pl.pallas_call(
        _nms_body,
        grid=(nb,),
        in_specs=[pl.BlockSpec((1, CB * 5, NP_OBJ), lambda g: (g, 0, 0)),
                  pl.BlockSpec((1, NP_OBJ, CB * 5), lambda g: (g, 0, 0))],
        out_specs=pl.BlockSpec((CB, 1, NP_OBJ), lambda g: (g, 0, 0)),
        out_shape=jax.ShapeDtypeStruct((nclass, 1, NP_OBJ), jnp.float32),
        scratch_shapes=[pltpu.VMEM((CB, NP_OBJ, NP_OBJ), jnp.bfloat16)],
    )(rowp, colp)


def _relmm_body(x_ref, w_ref, b_ref, o_ref):
    o_ref[...] = (
        jnp.dot(x_ref[...], w_ref[...], preferred_element_type=jnp.float32)
        + b_ref[...]
    )


def _relmm_call(vr, w, b):
    mb = 200
    return pl.pallas_call(
        _relmm_body,
        grid=(N_REL // mb,),
        in_specs=[
            pl.BlockSpec((mb, REL_DIM), lambda i: (i, 0)),
            pl.BlockSpec((REL_DIM, HID), lambda i: (0, 0)),
            pl.BlockSpec((1, HID), lambda i: (0, 0)),
        ],
        out_specs=pl.BlockSpec((mb, HID), lambda i: (i, 0)),
        out_shape=jax.ShapeDtypeStruct((N_REL, HID), jnp.float32),
    )(vr, w, b)


def _objmm_body(f_ref, w_ref, b_ref, lg_ref, mask_ref, probs_ref, wemb_ref,
                h_ref, pred_ref):
    lg = lg_ref[...]
    mask = mask_ref[...]
    lane = lax.broadcasted_iota(jnp.int32, lg.shape, 1)
    lp = mask * lg + (1.0 - mask) * (-1000.0)
    lp = jnp.where(lane < NUM_OBJ_CLS, lp, NEG)
    m = jnp.max(lp, axis=1, keepdims=True)
    e = jnp.exp(lp - m)
    p2 = e / jnp.sum(e, axis=1, keepdims=True)
    emb = jnp.dot(p2, wemb_ref[...], preferred_element_type=jnp.float32)
    h = jnp.dot(f_ref[...], w_ref[...], preferred_element_type=jnp.float32)
    h_ref[...] = jnp.tanh(h + b_ref[...] + emb)

    mp = mask * probs_ref[...]
    adj = jnp.where((lane >= 1) & (lane < NUM_OBJ_CLS), mp, -1.0)
    mx = jnp.max(adj, axis=1, keepdims=True)
    cand = jnp.where(adj == mx, lane, NUM_OBJ_CLS + 1)
    pred_ref[...] = jnp.min(cand, axis=1, keepdims=True)


def _objmm_call(fmaps, w, b, logits_p, mask_p, probs, wemb_p):
    mb = 200
    return pl.pallas_call(
        _objmm_body,
        grid=(N_OBJ // mb,),
        in_specs=[
            pl.BlockSpec((mb, OBJ_DIM), lambda i: (i, 0)),
            pl.BlockSpec((OBJ_DIM, HID), lambda i: (0, 0)),
            pl.BlockSpec((1, HID), lambda i: (0, 0)),
            pl.BlockSpec((mb, CP), lambda i: (i, 0)),
            pl.BlockSpec((mb, CP), lambda i: (i, 0)),
            pl.BlockSpec((mb, CP), lambda i: (i, 0)),
            pl.BlockSpec((CP, HID), lambda i: (0, 0)),
        ],
        out_specs=[
            pl.BlockSpec((mb, HID), lambda i: (i, 0)),
            pl.BlockSpec((mb, 1), lambda i: (i, 0)),
        ],
        out_shape=[
            jax.ShapeDtypeStruct((N_OBJ, HID), jnp.float32),
            jax.ShapeDtypeStruct((N_OBJ, 1), jnp.int32),
        ],
    )(fmaps, w, b, logits_p, mask_p, probs, wemb_p)


CHUNK = 80
NCHUNK = NP_REL // CHUNK
_NC = 2
_NS = 16
_ROWS_PER_SUB = NP_OBJ // _NS


def _sc_gather_body(objh_hbm, sub_hbm, ob_hbm, ga_hbm, gb_hbm,
                    idx_v, rows_v, sem):
    cid = lax.axis_index("c")
    sid = lax.axis_index("s")
    wid = sid * _NC + cid
    for h in range(2):
        k = wid * 2 + h
        pltpu.sync_copy(sub_hbm.at[k], idx_v)
        pltpu.async_copy(objh_hbm.at[idx_v], rows_v, sem).wait()
        pltpu.sync_copy(rows_v, ga_hbm.at[pl.ds(k * CHUNK, CHUNK)])
        pltpu.sync_copy(ob_hbm.at[k], idx_v)
        pltpu.async_copy(objh_hbm.at[idx_v], rows_v, sem).wait()
        pltpu.sync_copy(rows_v, gb_hbm.at[pl.ds(k * CHUNK, CHUNK)])


@functools.cache
def _sc_kernels():
    mesh = plsc.VectorSubcoreMesh(core_axis_name="c", subcore_axis_name="s",
                                  num_cores=_NC)
    gather = pl.kernel(
        _sc_gather_body,
        mesh=mesh,
        out_type=[jax.ShapeDtypeStruct((NP_REL, HID), jnp.float32),
                  jax.ShapeDtypeStruct((NP_REL, HID), jnp.float32)],
        scratch_types=[
            pltpu.VMEM((CHUNK,), jnp.int32),
            pltpu.VMEM((CHUNK, HID), jnp.float32),
            pltpu.SemaphoreType.DMA,
        ],
    )
    return gather


def _segsum(vr_get, subr_ref, obr_ref, objh, wmsg_ref):
    iota_obj_row = lax.broadcasted_iota(jnp.int32, (NP_OBJ, RB), 0)
    msg = jnp.zeros((NP_OBJ, HID), jnp.float32)
    for rb in range(NP_REL // RB):
        sub_blk = subr_ref[:, rb * RB:(rb + 1) * RB]
        ob_blk = obr_ref[:, rb * RB:(rb + 1) * RB]
        oh = ((sub_blk == iota_obj_row).astype(jnp.float32)
              + (ob_blk == iota_obj_row).astype(jnp.float32))
        msg = msg + jnp.dot(oh, vr_get(rb), preferred_element_type=jnp.float32)
    nrm = jnp.sqrt(jnp.sum(msg * msg, axis=1, keepdims=True))
    msg = msg / (nrm + 1e-8)
    return jnp.tanh(
        objh + jnp.dot(msg, wmsg_ref[...], preferred_element_type=jnp.float32))


def _seg0_body(vr_ref, subr_ref, obr_ref, objh_ref, wmsg_ref, out_ref):
    out_ref[...] = _segsum(
        lambda rb: vr_ref[rb * RB:(rb + 1) * RB, :],
        subr_ref, obr_ref, objh_ref[...], wmsg_ref)


def _seg0_call(vr, subr, obr, objh, wmsg):
    return pl.pallas_call(
        _seg0_body,
        out_shape=jax.ShapeDtypeStruct((NP_OBJ, HID), jnp.float32),
    )(vr, subr, obr, objh, wmsg)


def _segup_body(vr0_ref, ga_ref, gb_ref, subr_ref, obr_ref, objh_ref,
                wmsg_ref, objh_out, vr_out):
    for rb in range(NP_REL // RB):
        s = pl.ds(rb * RB, RB)
        vr_out[s, :] = jnp.tanh(vr0_ref[s, :] + ga_ref[s, :] + gb_ref[s, :])
    objh_out[...] = _segsum(
        lambda rb: vr_out[rb * RB:(rb + 1) * RB, :],
        subr_ref, obr_ref, objh_ref[...], wmsg_ref)


def _segup_call(vr0, ga, gb, subr, obr, objh, wmsg):
    return pl.pallas_call(
        _segup_body,
        out_shape=[jax.ShapeDtypeStruct((NP_OBJ, HID), jnp.float32),
                   jax.ShapeDtypeStruct((NP_REL, HID), jnp.float32)],
    )(vr0, ga, gb, subr, obr, objh, wmsg)


def _final_body(vr_ref, ga_ref, gb_ref, wout_ref, bout_ref, out_ref):
    ga = ga_ref[...]
    gb = gb_ref[...]
    vrn = jnp.tanh(vr_ref[...] + ga + gb)
    w1 = wout_ref[0:HID, :]
    w2 = wout_ref[HID:2 * HID, :]
    w3 = wout_ref[2 * HID:3 * HID, :]
    out_ref[...] = (
        jnp.dot(ga, w1, preferred_element_type=jnp.float32)
        + jnp.dot(gb, w2, preferred_element_type=jnp.float32)
        + jnp.dot(vrn, w3, preferred_element_type=jnp.float32)
        + bout_ref[...])


def _final_call(vr, ga, gb, wout_p, bout_p):
    blk = pl.BlockSpec((RB, HID), lambda i: (i, 0))
    return pl.pallas_call(
        _final_body,
        grid=(NP_REL // RB,),
        in_specs=[blk, blk, blk,
                  pl.BlockSpec((3 * HID, 128), lambda i: (0, 0)),
                  pl.BlockSpec((1, 128), lambda i: (0, 0))],
        out_specs=pl.BlockSpec((RB, 128), lambda i: (i, 0)),
        out_shape=jax.ShapeDtypeStruct((NP_REL, 128), jnp.float32),
    )(vr, ga, gb, wout_p, bout_p)


def kernel(im_inds, obj_fmaps, obj_logits, rel_inds, vr, boxes_per_cls,
           W_obj, b_obj, W_rel, b_rel, W_emb, W_msg, W_out, b_out):
    f32 = jnp.float32
    nclass = NUM_OBJ_CLS - 1

    logits_p = jnp.pad(obj_logits, ((0, 0), (0, CP - NUM_OBJ_CLS)),
                       constant_values=NEG)
    probs = _probs_call(logits_p)

    nb = nclass // CB
    bt = jnp.transpose(boxes_per_cls[:, 1:, :], (1, 2, 0))
    bt = jnp.pad(bt, ((0, 0), (0, 0), (0, NP_OBJ - N_OBJ)))
    st = jnp.transpose(probs[:, 1:NUM_OBJ_CLS])
    st = jnp.pad(st, ((0, 0), (0, NP_OBJ - N_OBJ)), constant_values=-1.0)
    arr5 = jnp.concatenate([bt, st[:, None, :]], axis=1)
    rowp = arr5.reshape(nb, CB * 5, NP_OBJ)
    colp = (arr5.reshape(nb, CB, 5, NP_OBJ)
            .transpose(0, 3, 1, 2).reshape(nb, NP_OBJ, CB * 5))
    keep = _nms_call(rowp, colp)
    keep2 = jnp.transpose(keep[:, 0, :N_OBJ])
    mask = jnp.concatenate([jnp.zeros((N_OBJ, 1), f32), keep2], axis=1)
    mask_p = jnp.pad(mask, ((0, 0), (0, CP - NUM_OBJ_CLS)))

    vr_h0 = _relmm_call(vr, W_rel, b_rel.reshape(1, HID))
    wemb_p = jnp.pad(W_emb, ((0, CP - NUM_OBJ_CLS), (0, 0)))
    obj_h0, preds = _objmm_call(obj_fmaps, W_obj, b_obj.reshape(1, HID),
                                logits_p, mask_p, probs, wemb_p)

    sub = rel_inds[:, 1]
    ob = rel_inds[:, 2]
    sub2 = jnp.pad(sub, (0, NP_REL - N_REL),
                   constant_values=NP_OBJ - 1).reshape(NCHUNK, CHUNK)
    ob2 = jnp.pad(ob, (0, NP_REL - N_REL),
                  constant_values=NP_OBJ - 1).reshape(NCHUNK, CHUNK)
    vr_hp = jnp.pad(vr_h0, ((0, NP_REL - N_REL), (0, 0)))
    objh_p = jnp.pad(obj_h0, ((0, NP_OBJ - N_OBJ), (0, 0)))
    wout_p = jnp.pad(W_out, ((0, 0), (0, 128 - NUM_REL_CLS)))
    bout_p = jnp.pad(b_out, (0, 128 - NUM_REL_CLS)).reshape(1, 128)

    sc_gather = _sc_kernels()
    subr = sub2.reshape(1, NP_REL)
    obr = ob2.reshape(1, NP_REL)
    objh_p = _seg0_call(vr_hp, subr, obr, objh_p, W_msg)
    for t in range(1, T_STEPS):
        ga, gb = sc_gather(objh_p, sub2, ob2)
        objh_p, vr_hp = _segup_call(vr_hp, ga, gb, subr, obr, objh_p, W_msg)
    ga, gb = sc_gather(objh_p, sub2, ob2)
    rel_p = _final_call(vr_hp, ga, gb, wout_p, bout_p)
    rel_logits = rel_p[:N_REL, :NUM_REL_CLS]

    obj_preds = preds.reshape(N_OBJ).astype(jnp.int32)
    return (obj_logits, obj_preds, rel_logits)


# speedup vs baseline: 1.1091x; 1.0052x over previous
import functools

import jax
import jax.numpy as jnp
from jax import lax
from jax.experimental import pallas as pl
from jax.experimental.pallas import tpu as pltpu
from jax.experimental.pallas import tpu_sc as plsc

N_OBJ = 1000
N_REL = 5000
NUM_OBJ_CLS = 151
NUM_REL_CLS = 51
OBJ_DIM = 4096
REL_DIM = 4096
HID = 512
T_STEPS = 3
NMS_THRESH = 0.3

NP_OBJ = 1024
CP = 256
NP_REL = 5120
RB = 512
NEG = -1e30


def _probs_body(logits_ref, out_ref):
    x = logits_ref[...]
    m = jnp.max(x, axis=1, keepdims=True)
    e = jnp.exp(x - m)
    out_ref[...] = e / jnp.sum(e, axis=1, keepdims=True)


def _probs_call(logits_p):
    return pl.pallas_call(
        _probs_body,
        out_shape=jax.ShapeDtypeStruct((N_OBJ, CP), jnp.float32),
    )(logits_p)


CB = 10


def _nms_body(rowp_ref, colp_ref, keep_ref, qs_ref):
    ii = lax.broadcasted_iota(jnp.int32, (NP_OBJ, NP_OBJ), 0)
    jj = lax.broadcasted_iota(jnp.int32, (NP_OBJ, NP_OBJ), 1)
    for c in range(CB):
        b = c * 5
        x1_r = rowp_ref[0, b + 0:b + 1, :]
        y1_r = rowp_ref[0, b + 1:b + 2, :]
        x2_r = rowp_ref[0, b + 2:b + 3, :]
        y2_r = rowp_ref[0, b + 3:b + 4, :]
        s_r = rowp_ref[0, b + 4:b + 5, :]
        x1_c = colp_ref[0, :, b + 0:b + 1]
        y1_c = colp_ref[0, :, b + 1:b + 2]
        x2_c = colp_ref[0, :, b + 2:b + 3]
        y2_c = colp_ref[0, :, b + 3:b + 4]
        s_c = colp_ref[0, :, b + 4:b + 5]

        area_r = (x2_r - x1_r) * (y2_r - y1_r)
        area_c = (x2_c - x1_c) * (y2_c - y1_c)
        xx1 = jnp.maximum(x1_c, x1_r)
        yy1 = jnp.maximum(y1_c, y1_r)
        xx2 = jnp.minimum(x2_c, x2_r)
        yy2 = jnp.minimum(y2_c, y2_r)
        w = jnp.maximum(xx2 - xx1, 0.0)
        h = jnp.maximum(yy2 - yy1, 0.0)
        inter = w * h
        iou = inter / (area_c + area_r - inter + 1e-8)

        prec = (s_c > s_r) | ((s_c == s_r) & (ii < jj))
        qs_ref[c] = ((iou > NMS_THRESH) & prec & (ii < N_OBJ) & (jj < N_OBJ)
                     ).astype(jnp.bfloat16)

    lane = lax.broadcasted_iota(jnp.int32, (CB, NP_OBJ), 1)
    valid = (lane < N_OBJ).astype(jnp.float32)

    def cond(state):
        _, _, und = state
        return jnp.sum(und) > 0.0

    def body(state):
        kept, supp, und = state
        notsupp = valid - supp
        new_kept_rows = []
        new_supp_rows = []
        for c in range(CB):
            lhs = jnp.concatenate(
                [kept[c:c + 1], notsupp[c:c + 1]], axis=0).astype(jnp.bfloat16)
            bp = jnp.dot(lhs, qs_ref[c], preferred_element_type=jnp.float32)
            new_supp_rows.append(und[c:c + 1] * (bp[0:1] > 0.0))
            new_kept_rows.append(und[c:c + 1] * (bp[1:2] == 0.0))
        new_supp = jnp.concatenate(new_supp_rows, axis=0)
        new_kept = jnp.concatenate(new_kept_rows, axis=0)
        kept = kept + new_kept
        supp = supp + new_supp
        und = und - new_kept - new_supp
        return kept, supp, und

    z = jnp.zeros((CB, NP_OBJ), jnp.float32)
    kept, _, _ = lax.while_loop(cond, body, (z, 1.0 - valid, valid))
    keep_ref[...] = kept[:, None, :]


def _nms_call(rowp, colp):
    nclass = NUM_OBJ_CLS - 1
    nb = nclass // CB
    return pl.pallas_call(
        _nms_body,
        grid=(nb,),
        in_specs=[pl.BlockSpec((1, CB * 5, NP_OBJ), lambda g: (g, 0, 0)),
                  pl.BlockSpec((1, NP_OBJ, CB * 5), lambda g: (g, 0, 0))],
        out_specs=pl.BlockSpec((CB, 1, NP_OBJ), lambda g: (g, 0, 0)),
        out_shape=jax.ShapeDtypeStruct((nclass, 1, NP_OBJ), jnp.float32),
        scratch_shapes=[pltpu.VMEM((CB, NP_OBJ, NP_OBJ), jnp.bfloat16)],
    )(rowp, colp)


def _relmm_body(x_ref, w_ref, b_ref, o_ref):
    o_ref[...] = (
        jnp.dot(x_ref[...], w_ref[...], preferred_element_type=jnp.float32)
        + b_ref[...]
    )


def _relmm_call(vr, w, b):
    mb = 200
    return pl.pallas_call(
        _relmm_body,
        grid=(N_REL // mb,),
        in_specs=[
            pl.BlockSpec((mb, REL_DIM), lambda i: (i, 0)),
            pl.BlockSpec((REL_DIM, HID), lambda i: (0, 0)),
            pl.BlockSpec((1, HID), lambda i: (0, 0)),
        ],
        out_specs=pl.BlockSpec((mb, HID), lambda i: (i, 0)),
        out_shape=jax.ShapeDtypeStruct((N_REL, HID), jnp.float32),
    )(vr, w, b)


def _objmm_body(f_ref, w_ref, b_ref, lg_ref, mask_ref, probs_ref, wemb_ref,
                h_ref, pred_ref):
    lg = lg_ref[...]
    mask = mask_ref[...]
    lane = lax.broadcasted_iota(jnp.int32, lg.shape, 1)
    lp = mask * lg + (1.0 - mask) * (-1000.0)
    lp = jnp.where(lane < NUM_OBJ_CLS, lp, NEG)
    m = jnp.max(lp, axis=1, keepdims=True)
    e = jnp.exp(lp - m)
    p2 = e / jnp.sum(e, axis=1, keepdims=True)
    emb = jnp.dot(p2, wemb_ref[...], preferred_element_type=jnp.float32)
    h = jnp.dot(f_ref[...], w_ref[...], preferred_element_type=jnp.float32)
    h_ref[...] = jnp.tanh(h + b_ref[...] + emb)

    mp = mask * probs_ref[...]
    adj = jnp.where((lane >= 1) & (lane < NUM_OBJ_CLS), mp, -1.0)
    mx = jnp.max(adj, axis=1, keepdims=True)
    cand = jnp.where(adj == mx, lane, NUM_OBJ_CLS + 1)
    pred_ref[...] = jnp.min(cand, axis=1, keepdims=True)


def _objmm_call(fmaps, w, b, logits_p, mask_p, probs, wemb_p):
    mb = 200
    return pl.pallas_call(
        _objmm_body,
        grid=(N_OBJ // mb,),
        in_specs=[
            pl.BlockSpec((mb, OBJ_DIM), lambda i: (i, 0)),
            pl.BlockSpec((OBJ_DIM, HID), lambda i: (0, 0)),
            pl.BlockSpec((1, HID), lambda i: (0, 0)),
            pl.BlockSpec((mb, CP), lambda i: (i, 0)),
            pl.BlockSpec((mb, CP), lambda i: (i, 0)),
            pl.BlockSpec((mb, CP), lambda i: (i, 0)),
            pl.BlockSpec((CP, HID), lambda i: (0, 0)),
        ],
        out_specs=[
            pl.BlockSpec((mb, HID), lambda i: (i, 0)),
            pl.BlockSpec((mb, 1), lambda i: (i, 0)),
        ],
        out_shape=[
            jax.ShapeDtypeStruct((N_OBJ, HID), jnp.float32),
            jax.ShapeDtypeStruct((N_OBJ, 1), jnp.int32),
        ],
    )(fmaps, w, b, logits_p, mask_p, probs, wemb_p)


CHUNK = 80
NCHUNK = NP_REL // CHUNK
_NC = 2
_NS = 16
_ROWS_PER_SUB = NP_OBJ // _NS


def _sc_gather_body(objh_hbm, sub_hbm, ob_hbm, ga_hbm, gb_hbm,
                    idx_a0, idx_b0, idx_a1, idx_b1,
                    rows_a, rows_b, sem_a, sem_b):
    cid = lax.axis_index("c")
    sid = lax.axis_index("s")
    wid = sid * _NC + cid
    k0 = wid * 2
    k1 = k0 + 1
    pltpu.sync_copy(sub_hbm.at[k0], idx_a0)
    pltpu.sync_copy(ob_hbm.at[k0], idx_b0)
    pltpu.sync_copy(sub_hbm.at[k1], idx_a1)
    pltpu.sync_copy(ob_hbm.at[k1], idx_b1)
    cp_a = pltpu.async_copy(objh_hbm.at[idx_a0], rows_a, sem_a)
    cp_b = pltpu.async_copy(objh_hbm.at[idx_b0], rows_b, sem_b)
    cp_a.wait()
    pltpu.sync_copy(rows_a, ga_hbm.at[pl.ds(k0 * CHUNK, CHUNK)])
    cp_a = pltpu.async_copy(objh_hbm.at[idx_a1], rows_a, sem_a)
    cp_b.wait()
    pltpu.sync_copy(rows_b, gb_hbm.at[pl.ds(k0 * CHUNK, CHUNK)])
    cp_b = pltpu.async_copy(objh_hbm.at[idx_b1], rows_b, sem_b)
    cp_a.wait()
    pltpu.sync_copy(rows_a, ga_hbm.at[pl.ds(k1 * CHUNK, CHUNK)])
    cp_b.wait()
    pltpu.sync_copy(rows_b, gb_hbm.at[pl.ds(k1 * CHUNK, CHUNK)])


@functools.cache
def _sc_kernels():
    mesh = plsc.VectorSubcoreMesh(core_axis_name="c", subcore_axis_name="s",
                                  num_cores=_NC)
    gather = pl.kernel(
        _sc_gather_body,
        mesh=mesh,
        out_type=[jax.ShapeDtypeStruct((NP_REL, HID), jnp.float32),
                  jax.ShapeDtypeStruct((NP_REL, HID), jnp.float32)],
        scratch_types=(
            [pltpu.VMEM((CHUNK,), jnp.int32)] * 4
            + [pltpu.VMEM((CHUNK, HID), jnp.float32)] * 2
            + [pltpu.SemaphoreType.DMA] * 2
        ),
    )
    return gather


def _segsum(vr_get, subr_ref, obr_ref, objh, wmsg_ref):
    iota_obj_row = lax.broadcasted_iota(jnp.int32, (NP_OBJ, RB), 0)
    msg = jnp.zeros((NP_OBJ, HID), jnp.float32)
    for rb in range(NP_REL // RB):
        sub_blk = subr_ref[:, rb * RB:(rb + 1) * RB]
        ob_blk = obr_ref[:, rb * RB:(rb + 1) * RB]
        oh = ((sub_blk == iota_obj_row).astype(jnp.float32)
              + (ob_blk == iota_obj_row).astype(jnp.float32))
        msg = msg + jnp.dot(oh, vr_get(rb), preferred_element_type=jnp.float32)
    nrm = jnp.sqrt(jnp.sum(msg * msg, axis=1, keepdims=True))
    msg = msg / (nrm + 1e-8)
    return jnp.tanh(
        objh + jnp.dot(msg, wmsg_ref[...], preferred_element_type=jnp.float32))


def _seg0_body(vr_ref, subr_ref, obr_ref, objh_ref, wmsg_ref, out_ref):
    out_ref[...] = _segsum(
        lambda rb: vr_ref[rb * RB:(rb + 1) * RB, :],
        subr_ref, obr_ref, objh_ref[...], wmsg_ref)


def _seg0_call(vr, subr, obr, objh, wmsg):
    return pl.pallas_call(
        _seg0_body,
        out_shape=jax.ShapeDtypeStruct((NP_OBJ, HID), jnp.float32),
    )(vr, subr, obr, objh, wmsg)


def _segup_body(vr0_ref, ga_ref, gb_ref, subr_ref, obr_ref, objh_ref,
                wmsg_ref, objh_out, vr_out):
    for rb in range(NP_REL // RB):
        s = pl.ds(rb * RB, RB)
        vr_out[s, :] = jnp.tanh(vr0_ref[s, :] + ga_ref[s, :] + gb_ref[s, :])
    objh_out[...] = _segsum(
        lambda rb: vr_out[rb * RB:(rb + 1) * RB, :],
        subr_ref, obr_ref, objh_ref[...], wmsg_ref)


def _segup_call(vr0, ga, gb, subr, obr, objh, wmsg):
    return pl.pallas_call(
        _segup_body,
        out_shape=[jax.ShapeDtypeStruct((NP_OBJ, HID), jnp.float32),
                   jax.ShapeDtypeStruct((NP_REL, HID), jnp.float32)],
    )(vr0, ga, gb, subr, obr, objh, wmsg)


def _final_body(vr_ref, ga_ref, gb_ref, wout_ref, bout_ref, out_ref):
    ga = ga_ref[...]
    gb = gb_ref[...]
    vrn = jnp.tanh(vr_ref[...] + ga + gb)
    w1 = wout_ref[0:HID, :]
    w2 = wout_ref[HID:2 * HID, :]
    w3 = wout_ref[2 * HID:3 * HID, :]
    out_ref[...] = (
        jnp.dot(ga, w1, preferred_element_type=jnp.float32)
        + jnp.dot(gb, w2, preferred_element_type=jnp.float32)
        + jnp.dot(vrn, w3, preferred_element_type=jnp.float32)
        + bout_ref[...])


def _final_call(vr, ga, gb, wout_p, bout_p):
    blk = pl.BlockSpec((RB, HID), lambda i: (i, 0))
    return pl.pallas_call(
        _final_body,
        grid=(NP_REL // RB,),
        in_specs=[blk, blk, blk,
                  pl.BlockSpec((3 * HID, 128), lambda i: (0, 0)),
                  pl.BlockSpec((1, 128), lambda i: (0, 0))],
        out_specs=pl.BlockSpec((RB, 128), lambda i: (i, 0)),
        out_shape=jax.ShapeDtypeStruct((NP_REL, 128), jnp.float32),
    )(vr, ga, gb, wout_p, bout_p)


def kernel(im_inds, obj_fmaps, obj_logits, rel_inds, vr, boxes_per_cls,
           W_obj, b_obj, W_rel, b_rel, W_emb, W_msg, W_out, b_out):
    f32 = jnp.float32
    nclass = NUM_OBJ_CLS - 1

    logits_p = jnp.pad(obj_logits, ((0, 0), (0, CP - NUM_OBJ_CLS)),
                       constant_values=NEG)
    probs = _probs_call(logits_p)

    nb = nclass // CB
    bt = jnp.transpose(boxes_per_cls[:, 1:, :], (1, 2, 0))
    bt = jnp.pad(bt, ((0, 0), (0, 0), (0, NP_OBJ - N_OBJ)))
    st = jnp.transpose(probs[:, 1:NUM_OBJ_CLS])
    st = jnp.pad(st, ((0, 0), (0, NP_OBJ - N_OBJ)), constant_values=-1.0)
    arr5 = jnp.concatenate([bt, st[:, None, :]], axis=1)
    rowp = arr5.reshape(nb, CB * 5, NP_OBJ)
    colp = (arr5.reshape(nb, CB, 5, NP_OBJ)
            .transpose(0, 3, 1, 2).reshape(nb, NP_OBJ, CB * 5))
    keep = _nms_call(rowp, colp)
    keep2 = jnp.transpose(keep[:, 0, :N_OBJ])
    mask = jnp.concatenate([jnp.zeros((N_OBJ, 1), f32), keep2], axis=1)
    mask_p = jnp.pad(mask, ((0, 0), (0, CP - NUM_OBJ_CLS)))

    vr_h0 = _relmm_call(vr, W_rel, b_rel.reshape(1, HID))
    wemb_p = jnp.pad(W_emb, ((0, CP - NUM_OBJ_CLS), (0, 0)))
    obj_h0, preds = _objmm_call(obj_fmaps, W_obj, b_obj.reshape(1, HID),
                                logits_p, mask_p, probs, wemb_p)

    sub = rel_inds[:, 1]
    ob = rel_inds[:, 2]
    sub2 = jnp.pad(sub, (0, NP_REL - N_REL),
                   constant_values=NP_OBJ - 1).reshape(NCHUNK, CHUNK)
    ob2 = jnp.pad(ob, (0, NP_REL - N_REL),
                  constant_values=NP_OBJ - 1).reshape(NCHUNK, CHUNK)
    vr_hp = jnp.pad(vr_h0, ((0, NP_REL - N_REL), (0, 0)))
    objh_p = jnp.pad(obj_h0, ((0, NP_OBJ - N_OBJ), (0, 0)))
    wout_p = jnp.pad(W_out, ((0, 0), (0, 128 - NUM_REL_CLS)))
    bout_p = jnp.pad(b_out, (0, 128 - NUM_REL_CLS)).reshape(1, 128)

    sc_gather = _sc_kernels()
    subr = sub2.reshape(1, NP_REL)
    obr = ob2.reshape(1, NP_REL)
    objh_p = _seg0_call(vr_hp, subr, obr, objh_p, W_msg)
    for t in range(1, T_STEPS):
        ga, gb = sc_gather(objh_p, sub2, ob2)
        objh_p, vr_hp = _segup_call(vr_hp, ga, gb, subr, obr, objh_p, W_msg)
    ga, gb = sc_gather(objh_p, sub2, ob2)
    rel_p = _final_call(vr_hp, ga, gb, wout_p, bout_p)
    rel_logits = rel_p[:N_REL, :NUM_REL_CLS]

    obj_preds = preds.reshape(N_OBJ).astype(jnp.int32)
    return (obj_logits, obj_preds, rel_logits)
